# baseline (speedup 1.0000x reference)
import functools
import math

import jax
import jax.numpy as jnp
import numpy as np
from jax import lax
from jax.experimental import pallas as pl
from jax.experimental.pallas import tpu as pltpu
from jax.experimental.pallas import tpu_sc as plsc

_N_BINS = 18
_EPS = 1e-9
_T = 1024
_LANES = 16
_CHUNKS = _T // _LANES
_N_ROWS = 9
_ROW_WORDS = _N_ROWS * _N_BINS
_LANE_STRIDE = 256
_TASK_WORDS = _LANES * _LANE_STRIDE
_N_TASKS = 128
_N_SBC = 16
_TASKS_PER_W = _N_TASKS // 32
_UNROLL = 4


def _sc_hist_kernel(x_hbm, lut_hbm, out_hbm,
                    pha_v, amp_v, hist_v, lo_v, hi_v, sem):
    nc = 2
    wid = lax.axis_index("s") * nc + lax.axis_index("c")
    base_task = wid * _TASKS_PER_W
    sbc = base_task // 8
    f_off = base_task - sbc * 8

    pltpu.sync_copy(lut_hbm.at[pl.ds(0, 24)], lo_v)
    pltpu.sync_copy(lut_hbm.at[pl.ds(24, 24)], hi_v)
    pltpu.sync_copy(x_hbm.at[pl.ds(sbc * 16 + 8, 8)], amp_v)
    pltpu.sync_copy(x_hbm.at[pl.ds(sbc * 16 + f_off, _TASKS_PER_W)], pha_v)

    lane_off = lax.iota(jnp.int32, _LANES) * _LANE_STRIDE
    scale = jnp.float32(_N_BINS / (2.0 * math.pi))
    pi32 = jnp.float32(math.pi)
    ones = jnp.full((_LANES,), 1.0, dtype=jnp.float32)
    zeros = jnp.zeros((_LANES,), dtype=jnp.float32)

    copies = []
    for tt in range(_TASKS_PER_W):
        t_off = tt * _TASK_WORDS

        @plsc.parallel_loop(0, _LANES, unroll=2)
        def _zero_body(i, t_off=t_off):
            off = t_off + i * _LANE_STRIDE
            for k in range(11):
                hist_v[pl.ds(off + k * _LANES, _LANES)] = zeros

        base_v = lane_off + t_off

        def _one_chunk(i, tt=tt, base_v=base_v):
            ph = pha_v[tt, pl.ds(i * _LANES, _LANES)]
            raw = (ph + pi32) * scale
            raw = jnp.minimum(jnp.maximum(raw, 0.0), 17.0)
            idx0 = raw.astype(jnp.int32)
            c_lo = plsc.load_gather(lo_v, [idx0])
            c_hi = plsc.load_gather(hi_v, [idx0])
            dec = (ph <= c_lo).astype(jnp.int32)
            inc = (ph > c_hi).astype(jnp.int32)
            base = base_v + (idx0 - dec + inc)
            for fa in range(8):
                av = amp_v[fa, pl.ds(i * _LANES, _LANES)]
                plsc.addupdate_scatter(hist_v, [base + fa * _N_BINS], av)
            plsc.addupdate_scatter(hist_v, [base + 8 * _N_BINS], ones)

        @plsc.parallel_loop(0, _CHUNKS, unroll=_UNROLL)
        def _chunk_body(j):
            _one_chunk(j)

        copies.append(pltpu.async_copy(
            hist_v.at[pl.ds(t_off, _TASK_WORDS)],
            out_hbm.at[base_task + tt], sem))
    for cp in copies:
        cp.wait()


def _sc_hist(x_t, lut):
    mesh = plsc.VectorSubcoreMesh(core_axis_name="c", subcore_axis_name="s")
    f = functools.partial(
        pl.kernel,
        mesh=mesh,
        out_type=jax.ShapeDtypeStruct((_N_TASKS, _TASK_WORDS), jnp.float32),
        scratch_types=[
            pltpu.VMEM((_TASKS_PER_W, _T), jnp.float32),
            pltpu.VMEM((8, _T), jnp.float32),
            pltpu.VMEM((_TASKS_PER_W * _TASK_WORDS,), jnp.float32),
            pltpu.VMEM((24,), jnp.float32),
            pltpu.VMEM((24,), jnp.float32),
            pltpu.SemaphoreType.DMA,
        ],
        compiler_params=pltpu.CompilerParams(needs_layout_passes=False),
    )(_sc_hist_kernel)
    return f(x_t, lut)


def _tc_finish_kernel(h_ref, o_ref, acc_ref):
    l = pl.program_id(0)

    @pl.when(l == 0)
    def _init():
        acc_ref[...] = h_ref[:, 0:_ROW_WORDS]

    @pl.when(l > 0)
    def _accum():
        acc_ref[...] += h_ref[:, 0:_ROW_WORDS]

    @pl.when(l == _LANES - 1)
    def _finish():
        _tc_mi(acc_ref[...], o_ref)


def _tc_mi(acc, o_ref):
    counts = acc[:, 8 * _N_BINS:9 * _N_BINS]
    log_n = jnp.float32(np.log(float(_N_BINS)))
    cols = []
    for fa in range(8):
        s_fa = acc[:, fa * _N_BINS:(fa + 1) * _N_BINS]
        mean = s_fa / (counts + _EPS)
        tot = jnp.sum(mean, axis=-1, keepdims=True)
        probs = mean / (tot + _EPS)
        kl = jnp.sum(probs * jnp.log(probs + _EPS), axis=-1, keepdims=True)
        cols.append((log_n + kl) / log_n)
    mi = jnp.concatenate(cols, axis=-1)
    mi = 0.5 * (mi[0:64, :] + mi[64:128, :])
    o_ref[...] = jnp.nan_to_num(mi, nan=0.0)


def _tc_finish(h2):
    return pl.pallas_call(
        _tc_finish_kernel,
        grid=(_LANES,),
        in_specs=[pl.BlockSpec((_N_TASKS, _LANE_STRIDE), lambda l: (0, l))],
        out_specs=pl.BlockSpec((64, 8), lambda l: (0, 0)),
        out_shape=jax.ShapeDtypeStruct((64, 8), jnp.float32),
        scratch_shapes=[pltpu.VMEM((_N_TASKS, _ROW_WORDS), jnp.float32)],
    )(h2)


def kernel(pha, amp):
    pha = pha.astype(jnp.float32)
    amp = amp.astype(jnp.float32)
    x_t = jnp.concatenate(
        [pha.transpose(3, 0, 1, 2, 4), amp.transpose(3, 0, 1, 2, 4)],
        axis=3).reshape(_N_SBC * 16, _T)
    cutoffs = np.linspace(-np.pi, np.pi, _N_BINS + 1).astype(np.float32)
    lo_t = np.full((24,), -np.inf, np.float32)
    lo_t[:18] = cutoffs[:18]
    lo_t[0] = -np.inf
    hi_t = np.full((24,), np.inf, np.float32)
    hi_t[:18] = cutoffs[1:19]
    hi_t[17] = np.inf
    lut = jnp.asarray(np.concatenate([lo_t, hi_t]))
    hist = _sc_hist(x_t, lut)
    mi = _tc_finish(hist)
    return mi.reshape(2, 4, 8, 8)


# speedup vs baseline: 1.3249x; 1.3249x over previous
import functools
import math

import jax
import jax.numpy as jnp
import numpy as np
from jax import lax
from jax.experimental import pallas as pl
from jax.experimental.pallas import tpu as pltpu
from jax.experimental.pallas import tpu_sc as plsc

_N_BINS = 18
_EPS = 1e-9
_T = 1024
_LANES = 16
_CHUNKS = _T // _LANES
_N_ROWS = 9
_ROW_WORDS = _N_ROWS * _N_BINS
_LANE_STRIDE = 256
_TASK_WORDS = _LANES * _LANE_STRIDE
_SC_TASKS = 64
_SC_TPW = _SC_TASKS // 32
_UNROLL = 4
_CUTS = np.linspace(-np.pi, np.pi, _N_BINS + 1).astype(np.float32)


def _sc_hist_kernel(x_hbm, lut_hbm, out_hbm,
                    pha_v, amp_v, hist_v, lo_v, hi_v, sem):
    nc = 2
    wid = lax.axis_index("s") * nc + lax.axis_index("c")
    base_task = wid * _SC_TPW
    sbc = base_task // 8
    f_off = base_task - sbc * 8

    pltpu.sync_copy(lut_hbm.at[pl.ds(0, 24)], lo_v)
    pltpu.sync_copy(lut_hbm.at[pl.ds(24, 24)], hi_v)
    pltpu.sync_copy(x_hbm.at[pl.ds(sbc * 16 + 8, 8)], amp_v)
    pltpu.sync_copy(x_hbm.at[pl.ds(sbc * 16 + f_off, _SC_TPW)], pha_v)

    lane_off = lax.iota(jnp.int32, _LANES) * _LANE_STRIDE
    scale = jnp.float32(_N_BINS / (2.0 * math.pi))
    pi32 = jnp.float32(math.pi)
    ones = jnp.full((_LANES,), 1.0, dtype=jnp.float32)
    zeros = jnp.zeros((_LANES,), dtype=jnp.float32)

    copies = []
    for tt in range(_SC_TPW):
        t_off = tt * _TASK_WORDS

        @plsc.parallel_loop(0, _LANES, unroll=2)
        def _zero_body(i, t_off=t_off):
            off = t_off + i * _LANE_STRIDE
            for k in range(11):
                hist_v[pl.ds(off + k * _LANES, _LANES)] = zeros

        base_v = lane_off + t_off

        def _one_chunk(i, tt=tt, base_v=base_v):
            ph = pha_v[tt, pl.ds(i * _LANES, _LANES)]
            raw = (ph + pi32) * scale
            raw = jnp.minimum(jnp.maximum(raw, 0.0), 17.0)
            idx0 = raw.astype(jnp.int32)
            c_lo = plsc.load_gather(lo_v, [idx0])
            c_hi = plsc.load_gather(hi_v, [idx0])
            dec = (ph <= c_lo).astype(jnp.int32)
            inc = (ph > c_hi).astype(jnp.int32)
            base = base_v + (idx0 - dec + inc)
            for fa in range(8):
                av = amp_v[fa, pl.ds(i * _LANES, _LANES)]
                plsc.addupdate_scatter(hist_v, [base + fa * _N_BINS], av)
            plsc.addupdate_scatter(hist_v, [base + 8 * _N_BINS], ones)

        @plsc.parallel_loop(0, _CHUNKS, unroll=_UNROLL)
        def _chunk_body(j):
            _one_chunk(j)

        copies.append(pltpu.async_copy(
            hist_v.at[pl.ds(t_off, _TASK_WORDS)],
            out_hbm.at[base_task + tt], sem))
    for cp in copies:
        cp.wait()


def _sc_hist(x_t, lut):
    mesh = plsc.VectorSubcoreMesh(core_axis_name="c", subcore_axis_name="s")
    f = functools.partial(
        pl.kernel,
        mesh=mesh,
        out_type=jax.ShapeDtypeStruct((_SC_TASKS, _TASK_WORDS), jnp.float32),
        scratch_types=[
            pltpu.VMEM((_SC_TPW, _T), jnp.float32),
            pltpu.VMEM((8, _T), jnp.float32),
            pltpu.VMEM((_SC_TPW * _TASK_WORDS,), jnp.float32),
            pltpu.VMEM((24,), jnp.float32),
            pltpu.VMEM((24,), jnp.float32),
            pltpu.SemaphoreType.DMA,
        ],
        compiler_params=pltpu.CompilerParams(needs_layout_passes=False),
    )(_sc_hist_kernel)
    return f(x_t, lut)


def _tc_half_kernel(x_ref, o_ref):
    x = x_ref[...]
    iota18 = lax.broadcasted_iota(jnp.int32, (_N_BINS, _T), 0).astype(
        jnp.float32)
    ones_row = jnp.ones((1, _T), jnp.float32)
    for sb in range(8):
        ph8 = x[sb * 16:sb * 16 + 8, :]
        am9 = jnp.concatenate([x[sb * 16 + 8:sb * 16 + 16, :], ones_row],
                              axis=0)
        bsum = jnp.zeros((8, _T), jnp.float32)
        for i in range(1, _N_BINS):
            bsum = bsum + jnp.where(ph8 > _CUTS[i], 1.0, 0.0)
        for fp in range(8):
            row = jnp.broadcast_to(bsum[fp:fp + 1, :], (_N_BINS, _T))
            mask = jnp.where(row == iota18, 1.0, 0.0)
            sums = lax.dot_general(
                am9, mask, (((1,), (1,)), ((), ())),
                preferred_element_type=jnp.float32)
            o_ref[sb * 8 + fp, :, :] = sums


def _tc_half(x_t):
    return pl.pallas_call(
        _tc_half_kernel,
        grid=(1,),
        in_specs=[pl.BlockSpec((128, _T), lambda i: (1, 0))],
        out_specs=pl.BlockSpec((64, _N_ROWS, _N_BINS), lambda i: (0, 0, 0)),
        out_shape=jax.ShapeDtypeStruct((64, _N_ROWS, _N_BINS), jnp.float32),
    )(x_t)


def _tc_finish_kernel(hsc_ref, htc_ref, o_ref):
    h = hsc_ref[...]
    acc = h[:, 0:_ROW_WORDS]
    for l in range(1, _LANES):
        off = l * _LANE_STRIDE
        acc = acc + h[:, off:off + _ROW_WORDS]
    htc = htc_ref[...]
    counts = jnp.concatenate(
        [acc[:, 8 * _N_BINS:9 * _N_BINS], htc[:, 8, :]], axis=0)
    log_n = jnp.float32(np.log(float(_N_BINS)))
    cols = []
    for fa in range(8):
        s_fa = jnp.concatenate(
            [acc[:, fa * _N_BINS:(fa + 1) * _N_BINS], htc[:, fa, :]], axis=0)
        mean = s_fa / (counts + _EPS)
        tot = jnp.sum(mean, axis=-1, keepdims=True)
        probs = mean / (tot + _EPS)
        kl = jnp.sum(probs * jnp.log(probs + _EPS), axis=-1, keepdims=True)
        cols.append((log_n + kl) / log_n)
    mi = jnp.concatenate(cols, axis=-1)
    mi = 0.5 * (mi[0:64, :] + mi[64:128, :])
    o_ref[...] = jnp.nan_to_num(mi, nan=0.0)


def _tc_finish(h_sc, h_tc):
    return pl.pallas_call(
        _tc_finish_kernel,
        out_shape=jax.ShapeDtypeStruct((64, 8), jnp.float32),
    )(h_sc, h_tc)


def kernel(pha, amp):
    pha = pha.astype(jnp.float32)
    amp = amp.astype(jnp.float32)
    x_t = jnp.concatenate(
        [pha.transpose(3, 0, 1, 2, 4), amp.transpose(3, 0, 1, 2, 4)],
        axis=3).reshape(256, _T)
    lo_t = np.full((24,), -np.inf, np.float32)
    lo_t[:18] = _CUTS[:18]
    lo_t[0] = -np.inf
    hi_t = np.full((24,), np.inf, np.float32)
    hi_t[:18] = _CUTS[1:19]
    hi_t[17] = np.inf
    lut = jnp.asarray(np.concatenate([lo_t, hi_t]))
    h_sc = _sc_hist(x_t, lut)
    h_tc = _tc_half(x_t)
    mi = _tc_finish(h_sc, h_tc)
    return mi.reshape(2, 4, 8, 8)


# speedup vs baseline: 1.3297x; 1.0036x over previous
import functools
import math

import jax
import jax.numpy as jnp
import numpy as np
from jax import lax
from jax.experimental import pallas as pl
from jax.experimental.pallas import tpu as pltpu
from jax.experimental.pallas import tpu_sc as plsc

_N_BINS = 18
_EPS = 1e-9
_T = 1024
_LANES = 16
_CHUNKS = _T // _LANES
_N_ROWS = 9
_ROW_WORDS = _N_ROWS * _N_BINS
_LANE_STRIDE = 256
_TASK_WORDS = _LANES * _LANE_STRIDE
_SC_TASKS = 64
_SC_TPW = _SC_TASKS // 32
_UNROLL = 4
_CUTS = np.linspace(-np.pi, np.pi, _N_BINS + 1).astype(np.float32)


def _sc_hist_kernel(x_hbm, lut_hbm, out_hbm,
                    pha_v, amp_v, hist_v, lo_v, hi_v, sem):
    nc = 2
    wid = lax.axis_index("s") * nc + lax.axis_index("c")
    base_task = wid * _SC_TPW
    sbc = base_task // 8
    f_off = base_task - sbc * 8

    pltpu.sync_copy(lut_hbm.at[pl.ds(0, 24)], lo_v)
    pltpu.sync_copy(lut_hbm.at[pl.ds(24, 24)], hi_v)
    pltpu.sync_copy(x_hbm.at[pl.ds(sbc * 16 + 8, 8)], amp_v)
    pltpu.sync_copy(x_hbm.at[pl.ds(sbc * 16 + f_off, _SC_TPW)], pha_v)

    lane_off = lax.iota(jnp.int32, _LANES) * _LANE_STRIDE
    scale = jnp.float32(_N_BINS / (2.0 * math.pi))
    pi32 = jnp.float32(math.pi)
    ones = jnp.full((_LANES,), 1.0, dtype=jnp.float32)
    zeros = jnp.zeros((_LANES,), dtype=jnp.float32)

    copies = []
    for tt in range(_SC_TPW):
        t_off = tt * _TASK_WORDS

        @plsc.parallel_loop(0, _LANES, unroll=2)
        def _zero_body(i, t_off=t_off):
            off = t_off + i * _LANE_STRIDE
            for k in range(11):
                hist_v[pl.ds(off + k * _LANES, _LANES)] = zeros

        base_v = lane_off + t_off

        def _one_chunk(i, tt=tt, base_v=base_v):
            ph = pha_v[tt, pl.ds(i * _LANES, _LANES)]
            raw = (ph + pi32) * scale
            raw = jnp.minimum(jnp.maximum(raw, 0.0), 17.0)
            idx0 = raw.astype(jnp.int32)
            c_lo = plsc.load_gather(lo_v, [idx0])
            c_hi = plsc.load_gather(hi_v, [idx0])
            dec = (ph <= c_lo).astype(jnp.int32)
            inc = (ph > c_hi).astype(jnp.int32)
            base = base_v + (idx0 - dec + inc)
            for fa in range(8):
                av = amp_v[fa, pl.ds(i * _LANES, _LANES)]
                plsc.addupdate_scatter(hist_v, [base + fa * _N_BINS], av)
            plsc.addupdate_scatter(hist_v, [base + 8 * _N_BINS], ones)

        @plsc.parallel_loop(0, _CHUNKS, unroll=_UNROLL)
        def _chunk_body(j):
            _one_chunk(j)

        copies.append(pltpu.async_copy(
            hist_v.at[pl.ds(t_off, _TASK_WORDS)],
            out_hbm.at[base_task + tt], sem))
    for cp in copies:
        cp.wait()


def _sc_hist(x_t, lut):
    mesh = plsc.VectorSubcoreMesh(core_axis_name="c", subcore_axis_name="s")
    f = functools.partial(
        pl.kernel,
        mesh=mesh,
        out_type=jax.ShapeDtypeStruct((_SC_TASKS, _TASK_WORDS), jnp.float32),
        scratch_types=[
            pltpu.VMEM((_SC_TPW, _T), jnp.float32),
            pltpu.VMEM((8, _T), jnp.float32),
            pltpu.VMEM((_SC_TPW * _TASK_WORDS,), jnp.float32),
            pltpu.VMEM((24,), jnp.float32),
            pltpu.VMEM((24,), jnp.float32),
            pltpu.SemaphoreType.DMA,
        ],
        compiler_params=pltpu.CompilerParams(needs_layout_passes=False),
    )(_sc_hist_kernel)
    return f(x_t, lut)


def _tc_half_kernel(x_ref, o_ref):
    x = x_ref[...]
    iota18 = lax.broadcasted_iota(jnp.int32, (_N_BINS, _T), 0).astype(
        jnp.float32)
    ones_row = jnp.ones((1, _T), jnp.float32)
    for sb in range(8):
        ph8 = x[sb * 16:sb * 16 + 8, :]
        am9 = jnp.concatenate([x[sb * 16 + 8:sb * 16 + 16, :], ones_row],
                              axis=0)
        bsum = jnp.zeros((8, _T), jnp.float32)
        for i in range(1, _N_BINS):
            bsum = bsum + jnp.where(ph8 > _CUTS[i], 1.0, 0.0)
        for fp in range(8):
            row = jnp.broadcast_to(bsum[fp:fp + 1, :], (_N_BINS, _T))
            mask = jnp.where(row == iota18, 1.0, 0.0)
            sums = lax.dot_general(
                am9, mask, (((1,), (1,)), ((), ())),
                precision=lax.Precision.HIGHEST,
                preferred_element_type=jnp.float32)
            o_ref[sb * 8 + fp, :, :] = sums


def _tc_half(x_t):
    return pl.pallas_call(
        _tc_half_kernel,
        grid=(1,),
        in_specs=[pl.BlockSpec((128, _T), lambda i: (1, 0))],
        out_specs=pl.BlockSpec((64, _N_ROWS, _N_BINS), lambda i: (0, 0, 0)),
        out_shape=jax.ShapeDtypeStruct((64, _N_ROWS, _N_BINS), jnp.float32),
    )(x_t)


def _tc_finish_kernel(hsc_ref, htc_ref, o_ref):
    h = hsc_ref[...]
    acc = h[:, 0:_ROW_WORDS]
    for l in range(1, _LANES):
        off = l * _LANE_STRIDE
        acc = acc + h[:, off:off + _ROW_WORDS]
    htc = htc_ref[...]
    counts = jnp.concatenate(
        [acc[:, 8 * _N_BINS:9 * _N_BINS], htc[:, 8, :]], axis=0)
    log_n = jnp.float32(np.log(float(_N_BINS)))
    cols = []
    for fa in range(8):
        s_fa = jnp.concatenate(
            [acc[:, fa * _N_BINS:(fa + 1) * _N_BINS], htc[:, fa, :]], axis=0)
        mean = s_fa / (counts + _EPS)
        tot = jnp.sum(mean, axis=-1, keepdims=True)
        probs = mean / (tot + _EPS)
        kl = jnp.sum(probs * jnp.log(probs + _EPS), axis=-1, keepdims=True)
        cols.append((log_n + kl) / log_n)
    mi = jnp.concatenate(cols, axis=-1)
    mi = 0.5 * (mi[0:64, :] + mi[64:128, :])
    o_ref[...] = jnp.nan_to_num(mi, nan=0.0)


def _tc_finish(h_sc, h_tc):
    return pl.pallas_call(
        _tc_finish_kernel,
        out_shape=jax.ShapeDtypeStruct((64, 8), jnp.float32),
    )(h_sc, h_tc)


def kernel(pha, amp):
    pha = pha.astype(jnp.float32)
    amp = amp.astype(jnp.float32)
    x_t = jnp.concatenate(
        [pha.transpose(3, 0, 1, 2, 4), amp.transpose(3, 0, 1, 2, 4)],
        axis=3).reshape(256, _T)
    lo_t = np.full((24,), -np.inf, np.float32)
    lo_t[:18] = _CUTS[:18]
    lo_t[0] = -np.inf
    hi_t = np.full((24,), np.inf, np.float32)
    hi_t[:18] = _CUTS[1:19]
    hi_t[17] = np.inf
    lut = jnp.asarray(np.concatenate([lo_t, hi_t]))
    h_sc = _sc_hist(x_t, lut)
    h_tc = _tc_half(x_t)
    mi = _tc_finish(h_sc, h_tc)
    return mi.reshape(2, 4, 8, 8)


# speedup vs baseline: 1.3812x; 1.0388x over previous
import functools
import math

import jax
import jax.numpy as jnp
import numpy as np
from jax import lax
from jax.experimental import pallas as pl
from jax.experimental.pallas import tpu as pltpu
from jax.experimental.pallas import tpu_sc as plsc

_N_BINS = 18
_EPS = 1e-9
_T = 1024
_LANES = 16
_CHUNKS = _T // _LANES
_N_ROWS = 9
_ROW_WORDS = _N_ROWS * _N_BINS
_LANE_STRIDE = 256
_TASK_WORDS = _LANES * _LANE_STRIDE
_SC_TASKS = 64
_SC_TPW = _SC_TASKS // 32
_UNROLL = 4
_CUTS = np.linspace(-np.pi, np.pi, _N_BINS + 1).astype(np.float32)


def _sc_hist_kernel(x_hbm, lut_hbm, out_hbm,
                    pha_v, amp_v, hist_v, lo_v, hi_v, sem):
    nc = 2
    wid = lax.axis_index("s") * nc + lax.axis_index("c")
    base_task = wid * _SC_TPW
    sbc = base_task // 8
    f_off = base_task - sbc * 8

    pltpu.sync_copy(lut_hbm.at[pl.ds(0, 24)], lo_v)
    pltpu.sync_copy(lut_hbm.at[pl.ds(24, 24)], hi_v)
    pltpu.sync_copy(x_hbm.at[pl.ds(sbc * 16 + 8, 8)], amp_v)
    pltpu.sync_copy(x_hbm.at[pl.ds(sbc * 16 + f_off, _SC_TPW)], pha_v)

    lane_off = lax.iota(jnp.int32, _LANES) * _LANE_STRIDE
    scale = jnp.float32(_N_BINS / (2.0 * math.pi))
    pi32 = jnp.float32(math.pi)
    ones = jnp.full((_LANES,), 1.0, dtype=jnp.float32)
    zeros = jnp.zeros((_LANES,), dtype=jnp.float32)

    copies = []
    for tt in range(_SC_TPW):
        t_off = tt * _TASK_WORDS

        @plsc.parallel_loop(0, _LANES, unroll=2)
        def _zero_body(i, t_off=t_off):
            off = t_off + i * _LANE_STRIDE
            for k in range(11):
                hist_v[pl.ds(off + k * _LANES, _LANES)] = zeros

        base_v = lane_off + t_off

        def _one_chunk(i, tt=tt, base_v=base_v):
            ph = pha_v[tt, pl.ds(i * _LANES, _LANES)]
            raw = (ph + pi32) * scale
            raw = jnp.minimum(jnp.maximum(raw, 0.0), 17.0)
            idx0 = raw.astype(jnp.int32)
            c_lo = plsc.load_gather(lo_v, [idx0])
            c_hi = plsc.load_gather(hi_v, [idx0])
            dec = (ph <= c_lo).astype(jnp.int32)
            inc = (ph > c_hi).astype(jnp.int32)
            base = base_v + (idx0 - dec + inc)
            for fa in range(8):
                av = amp_v[fa, pl.ds(i * _LANES, _LANES)]
                plsc.addupdate_scatter(hist_v, [base + fa * _N_BINS], av)
            plsc.addupdate_scatter(hist_v, [base + 8 * _N_BINS], ones)

        @plsc.parallel_loop(0, _CHUNKS, unroll=_UNROLL)
        def _chunk_body(j):
            _one_chunk(j)

        copies.append(pltpu.async_copy(
            hist_v.at[pl.ds(t_off, _TASK_WORDS)],
            out_hbm.at[base_task + tt], sem))
    for cp in copies:
        cp.wait()


def _sc_hist(x_t, lut):
    mesh = plsc.VectorSubcoreMesh(core_axis_name="c", subcore_axis_name="s")
    f = functools.partial(
        pl.kernel,
        mesh=mesh,
        out_type=jax.ShapeDtypeStruct((_SC_TASKS, _TASK_WORDS), jnp.float32),
        scratch_types=[
            pltpu.VMEM((_SC_TPW, _T), jnp.float32),
            pltpu.VMEM((8, _T), jnp.float32),
            pltpu.VMEM((_SC_TPW * _TASK_WORDS,), jnp.float32),
            pltpu.VMEM((24,), jnp.float32),
            pltpu.VMEM((24,), jnp.float32),
            pltpu.SemaphoreType.DMA,
        ],
        compiler_params=pltpu.CompilerParams(needs_layout_passes=False),
    )(_sc_hist_kernel)
    return f(x_t, lut)


def _tc_half_kernel(x_ref, o_ref):
    x = x_ref[...]
    iota18 = lax.broadcasted_iota(jnp.int32, (_N_BINS, _T), 0).astype(
        jnp.float32)
    ones_row = jnp.ones((1, _T), jnp.float32)
    for sb in range(8):
        ph8 = x[sb * 16:sb * 16 + 8, :]
        am9 = jnp.concatenate([x[sb * 16 + 8:sb * 16 + 16, :], ones_row],
                              axis=0)
        bsum = jnp.zeros((8, _T), jnp.float32)
        for i in range(1, _N_BINS):
            bsum = bsum + jnp.where(ph8 > _CUTS[i], 1.0, 0.0)
        for fp in range(8):
            row = jnp.broadcast_to(bsum[fp:fp + 1, :], (_N_BINS, _T))
            mask = jnp.where(row == iota18, 1.0, 0.0)
            sums = lax.dot_general(
                am9, mask, (((1,), (1,)), ((), ())),
                precision=lax.Precision.HIGHEST,
                preferred_element_type=jnp.float32)
            mi = _mi_from_sums(sums[:8, :], sums[8:9, :])
            o_ref[sb * 8 + fp, :] = mi[:, 0]


def _mi_from_sums(s_fa, counts):
    log_n = jnp.float32(np.log(float(_N_BINS)))
    mean = s_fa / (counts + _EPS)
    tot = jnp.sum(mean, axis=-1, keepdims=True)
    probs = mean / (tot + _EPS)
    kl = jnp.sum(probs * jnp.log(probs + _EPS), axis=-1, keepdims=True)
    return (log_n + kl) / log_n


def _tc_half(x_t):
    return pl.pallas_call(
        _tc_half_kernel,
        grid=(1,),
        in_specs=[pl.BlockSpec((128, _T), lambda i: (1, 0))],
        out_specs=pl.BlockSpec((64, 8), lambda i: (0, 0)),
        out_shape=jax.ShapeDtypeStruct((64, 8), jnp.float32),
    )(x_t)


def _tc_finish_kernel(hsc_ref, mitc_ref, o_ref):
    h = hsc_ref[...]
    acc = h[:, 0:_ROW_WORDS]
    for l in range(1, _LANES):
        off = l * _LANE_STRIDE
        acc = acc + h[:, off:off + _ROW_WORDS]
    counts = acc[:, 8 * _N_BINS:9 * _N_BINS]
    cols = [_mi_from_sums(acc[:, fa * _N_BINS:(fa + 1) * _N_BINS], counts)
            for fa in range(8)]
    mi_sc = jnp.concatenate(cols, axis=-1)
    mi = 0.5 * (mi_sc + mitc_ref[...])
    o_ref[...] = jnp.nan_to_num(mi, nan=0.0)


def _tc_finish(h_sc, h_tc):
    return pl.pallas_call(
        _tc_finish_kernel,
        out_shape=jax.ShapeDtypeStruct((64, 8), jnp.float32),
    )(h_sc, h_tc)


def kernel(pha, amp):
    pha = pha.astype(jnp.float32)
    amp = amp.astype(jnp.float32)
    x_t = jnp.concatenate(
        [pha.transpose(3, 0, 1, 2, 4), amp.transpose(3, 0, 1, 2, 4)],
        axis=3).reshape(256, _T)
    lo_t = np.full((24,), -np.inf, np.float32)
    lo_t[:18] = _CUTS[:18]
    lo_t[0] = -np.inf
    hi_t = np.full((24,), np.inf, np.float32)
    hi_t[:18] = _CUTS[1:19]
    hi_t[17] = np.inf
    lut = jnp.asarray(np.concatenate([lo_t, hi_t]))
    h_sc = _sc_hist(x_t, lut)
    h_tc = _tc_half(x_t)
    mi = _tc_finish(h_sc, h_tc)
    return mi.reshape(2, 4, 8, 8)
